# baseline (speedup 1.0000x reference)
import functools

import jax
import jax.numpy as jnp
from jax import lax
from jax.experimental import pallas as pl
from jax.experimental.pallas import tpu as pltpu
from jax.experimental.pallas import tpu_sc as plsc

BATCH = 16384
HIST = 20
EMBED_DIM = 32
NUM_ROWS = BATCH * HIST
MAX_ID = 1000000
NC, NS = 2, 16
NW = NC * NS
B_PER_W = BATCH // NW
ROWS_PER_W = NUM_ROWS // NW
CHUNK = 1024
N_CHUNKS = ROWS_PER_W // CHUNK
LANES = 16


def _gather_body(idxt_hbm, table_hbm, out_hbm, ibuf, idx_v, rows0, rows1,
                 sem_i, sem_g, sem_o):
    wid = lax.axis_index("s") * NC + lax.axis_index("c")
    base = wid * ROWS_PER_W
    b0 = wid * B_PER_W

    pltpu.async_copy(idxt_hbm.at[:, pl.ds(b0, B_PER_W)], ibuf, sem_i).wait()
    iota_h = lax.iota(jnp.int32, LANES) * HIST
    for h in range(HIST):
        def row_step(j, _, h=h):
            r0 = j * LANES
            vec = ibuf[h, pl.ds(r0, LANES)]
            plsc.store_scatter(idx_v, [iota_h + (r0 * HIST + h)], vec)
            return _

        lax.fori_loop(0, B_PER_W // LANES, row_step, 0, unroll=4)

    bufs = (rows0, rows1)
    gathers = [None] * N_CHUNKS
    writes = [None] * N_CHUNKS
    for g in range(N_CHUNKS):
        gathers[g] = pltpu.async_copy(
            table_hbm.at[idx_v.at[pl.ds(g * CHUNK, CHUNK)]], bufs[g % 2],
            sem_g)
        if g >= 1:
            if g >= 2:
                writes[g - 2].wait()
            gathers[g - 1].wait()
            writes[g - 1] = pltpu.async_copy(
                bufs[(g - 1) % 2],
                out_hbm.at[pl.ds(base + (g - 1) * CHUNK, CHUNK)], sem_o)
    gathers[N_CHUNKS - 1].wait()
    writes[N_CHUNKS - 2].wait()
    writes[N_CHUNKS - 1] = pltpu.async_copy(
        bufs[(N_CHUNKS - 1) % 2],
        out_hbm.at[pl.ds(base + (N_CHUNKS - 1) * CHUNK, CHUNK)], sem_o)
    writes[N_CHUNKS - 1].wait()


def _kernel_impl(indices, table):
    idx_t = indices.astype(jnp.int32).T
    mesh = plsc.VectorSubcoreMesh(
        core_axis_name="c", subcore_axis_name="s",
        num_cores=NC, num_subcores=NS,
    )
    run = pl.kernel(
        _gather_body,
        out_type=jax.ShapeDtypeStruct((NUM_ROWS, EMBED_DIM), jnp.float32),
        mesh=mesh,
        scratch_types=[
            pltpu.VMEM((HIST, B_PER_W), jnp.int32),
            pltpu.VMEM((ROWS_PER_W,), jnp.int32),
            pltpu.VMEM((CHUNK, EMBED_DIM), jnp.float32),
            pltpu.VMEM((CHUNK, EMBED_DIM), jnp.float32),
            pltpu.SemaphoreType.DMA,
            pltpu.SemaphoreType.DMA,
            pltpu.SemaphoreType.DMA,
        ],
        compiler_params=pltpu.CompilerParams(
            use_tc_tiling_on_sc=False, needs_layout_passes=False),
    )
    out = run(idx_t, table)
    return out.reshape(BATCH, HIST, EMBED_DIM)


kernel = jax.jit(_kernel_impl)
